# baseline (speedup 1.0000x reference)
import functools

import jax
import jax.numpy as jnp
from jax import lax
from jax.experimental import pallas as pl
from jax.experimental.pallas import tpu as pltpu
from jax.experimental.pallas import tpu_sc as plsc

DIM = 128
BATCH = 16384
NW = 32
CHUNK = 128
ROWS_PER_W = BATCH // NW
NCHUNK = ROWS_PER_W // CHUNK
NBLK = CHUNK // 16


def _rsqrt(x):
    i = lax.bitcast_convert_type(x, jnp.int32)
    i = jnp.int32(0x5F3759DF) - lax.shift_right_arithmetic(i, 1)
    y = lax.bitcast_convert_type(i, jnp.float32)
    for _ in range(3):
        y = y * (1.5 - 0.5 * x * y * y)
    return y


def _make_kernel():
    mesh = plsc.VectorSubcoreMesh(core_axis_name="c", subcore_axis_name="s")

    @functools.partial(
        pl.kernel,
        mesh=mesh,
        compiler_params=pltpu.CompilerParams(needs_layout_passes=False),
        out_type=jax.ShapeDtypeStruct((NW, NCHUNK, CHUNK), jnp.float32),
        scratch_types=[
            pltpu.VMEM((NCHUNK, CHUNK), jnp.int32),
            pltpu.VMEM((NCHUNK, CHUNK), jnp.int32),
            pltpu.VMEM((NCHUNK, CHUNK), jnp.int32),
            pltpu.VMEM((CHUNK, DIM), jnp.float32),
            pltpu.VMEM((CHUNK, DIM), jnp.float32),
            pltpu.VMEM((CHUNK, DIM), jnp.float32),
            pltpu.VMEM((NCHUNK, CHUNK), jnp.float32),
            pltpu.SemaphoreType.DMA,
            pltpu.SemaphoreType.DMA,
            pltpu.SemaphoreType.DMA,
        ],
    )
    def trans_score(h_hbm, t_hbm, r_hbm, tail_hbm, rel_hbm, out_hbm,
                    hidx, tidx, ridx, hbuf, tbuf, rbuf, score,
                    sem_h, sem_t, sem_r):
        wid = lax.axis_index("s") * 2 + lax.axis_index("c")
        pltpu.sync_copy(h_hbm.at[wid], hidx)
        pltpu.sync_copy(t_hbm.at[wid], tidx)
        pltpu.sync_copy(r_hbm.at[wid], ridx)

        lane = lax.iota(jnp.int32, 16)
        zero = jnp.zeros((16,), jnp.float32)

        for c in range(NCHUNK):
            dh = pltpu.async_copy(tail_hbm.at[hidx.at[c]], hbuf, sem_h)
            dt = pltpu.async_copy(tail_hbm.at[tidx.at[c]], tbuf, sem_t)
            dr = pltpu.async_copy(rel_hbm.at[ridx.at[c]], rbuf, sem_r)
            dh.wait()
            dt.wait()
            dr.wait()

            def block_fn(b, _, c=c):
                rows = b * 16 + lane

                def p1(j, accs):
                    ah, at_, ar = accs
                    jv = jnp.zeros((16,), jnp.int32) + j
                    hj = plsc.load_gather(hbuf, [rows, jv])
                    tj = plsc.load_gather(tbuf, [rows, jv])
                    rj = plsc.load_gather(rbuf, [rows, jv])
                    return (ah + hj * hj, at_ + tj * tj, ar + rj * rj)

                sh, st, sr = lax.fori_loop(0, DIM, p1, (zero, zero, zero),
                                           unroll=8)
                ih = _rsqrt(jnp.maximum(sh, 1e-24))
                it = _rsqrt(jnp.maximum(st, 1e-24))
                ir = _rsqrt(jnp.maximum(sr, 1e-24))

                def p2(j, acc):
                    jv = jnp.zeros((16,), jnp.int32) + j
                    hj = plsc.load_gather(hbuf, [rows, jv])
                    tj = plsc.load_gather(tbuf, [rows, jv])
                    rj = plsc.load_gather(rbuf, [rows, jv])
                    return acc + jnp.abs(hj * ih + rj * ir - tj * it)

                sc = lax.fori_loop(0, DIM, p2, zero, unroll=8)
                score[c, pl.ds(b * 16, 16)] = sc
                return 0

            lax.fori_loop(0, NBLK, block_fn, 0)

        pltpu.sync_copy(score, out_hbm.at[wid])

    return trans_score


_sc_score = _make_kernel()


def kernel(batch_h, batch_t, batch_r, tail_emb, rel_emb):
    h = batch_h.astype(jnp.int32).reshape(NW, NCHUNK, CHUNK)
    t = batch_t.astype(jnp.int32).reshape(NW, NCHUNK, CHUNK)
    r = batch_r.astype(jnp.int32).reshape(NW, NCHUNK, CHUNK)
    out = _sc_score(h, t, r, tail_emb, rel_emb)
    return out.reshape(-1)


# speedup vs baseline: 2.8180x; 2.8180x over previous
import functools

import jax
import jax.numpy as jnp
from jax import lax
from jax.experimental import pallas as pl
from jax.experimental.pallas import tpu as pltpu
from jax.experimental.pallas import tpu_sc as plsc

DIM = 128
BATCH = 16384
NW = 32
CHUNK = 128
ROWS_PER_W = BATCH // NW
NCHUNK = ROWS_PER_W // CHUNK
NBLK = CHUNK // 16
NSUB = DIM // 16


def _rsqrt(x):
    i = lax.bitcast_convert_type(x, jnp.int32)
    i = jnp.int32(0x5F3759DF) - lax.shift_right_arithmetic(i, 1)
    y = lax.bitcast_convert_type(i, jnp.float32)
    for _ in range(3):
        y = y * (1.5 - 0.5 * x * y * y)
    return y


def _make_kernel():
    mesh = plsc.VectorSubcoreMesh(core_axis_name="c", subcore_axis_name="s")

    @functools.partial(
        pl.kernel,
        mesh=mesh,
        compiler_params=pltpu.CompilerParams(needs_layout_passes=False),
        out_type=jax.ShapeDtypeStruct((NW, NCHUNK, CHUNK), jnp.float32),
        scratch_types=[
            pltpu.VMEM((NCHUNK, CHUNK), jnp.int32),
            pltpu.VMEM((NCHUNK, CHUNK), jnp.int32),
            pltpu.VMEM((NCHUNK, CHUNK), jnp.int32),
            pltpu.VMEM((2, CHUNK, DIM), jnp.float32),
            pltpu.VMEM((2, CHUNK, DIM), jnp.float32),
            pltpu.VMEM((2, CHUNK, DIM), jnp.float32),
            pltpu.VMEM((16, 17), jnp.float32),
            pltpu.VMEM((16, 17), jnp.float32),
            pltpu.VMEM((16, 17), jnp.float32),
            pltpu.VMEM((16, 17), jnp.float32),
            pltpu.VMEM((NCHUNK, CHUNK), jnp.float32),
            pltpu.SemaphoreType.DMA,
            pltpu.SemaphoreType.DMA,
            pltpu.SemaphoreType.DMA,
            pltpu.SemaphoreType.DMA,
            pltpu.SemaphoreType.DMA,
            pltpu.SemaphoreType.DMA,
        ],
    )
    def trans_score(h_hbm, t_hbm, r_hbm, tail_hbm, rel_hbm, out_hbm,
                    hidx, tidx, ridx, hbuf, tbuf, rbuf,
                    pb_h, pb_t, pb_r, pb_s, score,
                    sh0, st0, sr0, sh1, st1, sr1):
        wid = lax.axis_index("s") * 2 + lax.axis_index("c")
        pltpu.sync_copy(h_hbm.at[wid], hidx)
        pltpu.sync_copy(t_hbm.at[wid], tidx)
        pltpu.sync_copy(r_hbm.at[wid], ridx)

        lane = lax.iota(jnp.int32, 16)
        zero = jnp.zeros((16,), jnp.float32)
        sems = ((sh0, st0, sr0), (sh1, st1, sr1))

        def fire(c, p):
            dh = pltpu.async_copy(tail_hbm.at[hidx.at[c]], hbuf.at[p],
                                  sems[p][0])
            dt = pltpu.async_copy(tail_hbm.at[tidx.at[c]], tbuf.at[p],
                                  sems[p][1])
            dr = pltpu.async_copy(rel_hbm.at[ridx.at[c]], rbuf.at[p],
                                  sems[p][2])
            return (dh, dt, dr)

        pend = fire(0, 0)
        for c in range(NCHUNK):
            p = c % 2
            for d in pend:
                d.wait()
            if c + 1 < NCHUNK:
                pend = fire(c + 1, 1 - p)
            hb, tb, rb = hbuf.at[p], tbuf.at[p], rbuf.at[p]

            def block_fn(b, _, hb=hb, tb=tb, rb=rb, c=c):
                base = b * 16

                def pass1(i, _):
                    row = base + i
                    ph = pt = pr = zero
                    for s in range(NSUB):
                        sl = pl.ds(s * 16, 16)
                        hv = hb[row, sl]
                        tv = tb[row, sl]
                        rv = rb[row, sl]
                        ph = ph + hv * hv
                        pt = pt + tv * tv
                        pr = pr + rv * rv
                    pb_h[i, pl.ds(0, 16)] = ph
                    pb_t[i, pl.ds(0, 16)] = pt
                    pb_r[i, pl.ds(0, 16)] = pr
                    return 0

                lax.fori_loop(0, 16, pass1, 0, unroll=2)

                sh = st = sr = zero
                for j in range(16):
                    jv = jnp.full((16,), j, jnp.int32)
                    sh = sh + plsc.load_gather(pb_h, [lane, jv])
                    st = st + plsc.load_gather(pb_t, [lane, jv])
                    sr = sr + plsc.load_gather(pb_r, [lane, jv])
                ih = _rsqrt(jnp.maximum(sh, 1e-24))
                it = _rsqrt(jnp.maximum(st, 1e-24))
                ir = _rsqrt(jnp.maximum(sr, 1e-24))
                for j in range(16):
                    jv = jnp.full((16,), j, jnp.int32)
                    plsc.store_scatter(pb_h, [lane, jv], ih)
                    plsc.store_scatter(pb_t, [lane, jv], it)
                    plsc.store_scatter(pb_r, [lane, jv], ir)

                def pass2(i, _):
                    row = base + i
                    ihv = pb_h[i, pl.ds(0, 16)]
                    itv = pb_t[i, pl.ds(0, 16)]
                    irv = pb_r[i, pl.ds(0, 16)]
                    acc = zero
                    for s in range(NSUB):
                        sl = pl.ds(s * 16, 16)
                        acc = acc + jnp.abs(hb[row, sl] * ihv
                                            + rb[row, sl] * irv
                                            - tb[row, sl] * itv)
                    pb_s[i, pl.ds(0, 16)] = acc
                    return 0

                lax.fori_loop(0, 16, pass2, 0, unroll=2)

                sc = zero
                for j in range(16):
                    jv = jnp.full((16,), j, jnp.int32)
                    sc = sc + plsc.load_gather(pb_s, [lane, jv])
                score[c, pl.ds(base, 16)] = sc
                return 0

            lax.fori_loop(0, NBLK, block_fn, 0)

        pltpu.sync_copy(score, out_hbm.at[wid])

    return trans_score


_sc_score = _make_kernel()


def kernel(batch_h, batch_t, batch_r, tail_emb, rel_emb):
    h = batch_h.astype(jnp.int32).reshape(NW, NCHUNK, CHUNK)
    t = batch_t.astype(jnp.int32).reshape(NW, NCHUNK, CHUNK)
    r = batch_r.astype(jnp.int32).reshape(NW, NCHUNK, CHUNK)
    out = _sc_score(h, t, r, tail_emb, rel_emb)
    return out.reshape(-1)


# speedup vs baseline: 4.6978x; 1.6671x over previous
import functools

import jax
import jax.numpy as jnp
from jax import lax
from jax.experimental import pallas as pl
from jax.experimental.pallas import tpu as pltpu
from jax.experimental.pallas import tpu_sc as plsc

DIM = 128
BATCH = 16384
NW = 32
CHUNK = 128
ROWS_PER_W = BATCH // NW
NCHUNK = ROWS_PER_W // CHUNK
NBLK = CHUNK // 16
NSUB = DIM // 16


def _treesum(xs):
    xs = list(xs)
    while len(xs) > 1:
        nxt = [a + b for a, b in zip(xs[::2], xs[1::2])]
        if len(xs) % 2:
            nxt.append(xs[-1])
        xs = nxt
    return xs[0]


def _rsqrt(x):
    x = jnp.maximum(x, jnp.float32(1e-24))
    i = lax.bitcast_convert_type(x, jnp.int32)
    i = jnp.int32(0x5F3759DF) - lax.shift_right_arithmetic(i, 1)
    y = lax.bitcast_convert_type(i, jnp.float32)
    xh = jnp.float32(0.5) * x
    for _ in range(3):
        y = y * (jnp.float32(1.5) - xh * y * y)
    return y


def _make_kernel():
    mesh = plsc.VectorSubcoreMesh(core_axis_name="c", subcore_axis_name="s")

    @functools.partial(
        pl.kernel,
        mesh=mesh,
        compiler_params=pltpu.CompilerParams(needs_layout_passes=False),
        out_type=jax.ShapeDtypeStruct((NW, NCHUNK, CHUNK), jnp.float32),
        scratch_types=[
            pltpu.VMEM((NCHUNK, CHUNK), jnp.int32),
            pltpu.VMEM((NCHUNK, CHUNK), jnp.int32),
            pltpu.VMEM((NCHUNK, CHUNK), jnp.int32),
            pltpu.VMEM((2, CHUNK, DIM), jnp.float32),
            pltpu.VMEM((2, CHUNK, DIM), jnp.float32),
            pltpu.VMEM((2, CHUNK, DIM), jnp.float32),
            pltpu.VMEM((CHUNK, 17), jnp.float32),
            pltpu.VMEM((NCHUNK, CHUNK), jnp.float32),
            pltpu.SemaphoreType.DMA,
            pltpu.SemaphoreType.DMA,
            pltpu.SemaphoreType.DMA,
            pltpu.SemaphoreType.DMA,
            pltpu.SemaphoreType.DMA,
            pltpu.SemaphoreType.DMA,
        ],
    )
    def trans_score(h_hbm, t_hbm, r_hbm, tail_hbm, rel_hbm, out_hbm,
                    hidx, tidx, ridx, hbuf, tbuf, rbuf, pb_s, score,
                    sh0, st0, sr0, sh1, st1, sr1):
        wid = lax.axis_index("s") * 2 + lax.axis_index("c")
        pltpu.sync_copy(h_hbm.at[wid], hidx)
        pltpu.sync_copy(t_hbm.at[wid], tidx)
        pltpu.sync_copy(r_hbm.at[wid], ridx)

        lane = lax.iota(jnp.int32, 16)
        zero = jnp.zeros((16,), jnp.float32)
        sems = ((sh0, st0, sr0), (sh1, st1, sr1))

        def fire(c, p):
            dh = pltpu.async_copy(tail_hbm.at[hidx.at[c]], hbuf.at[p],
                                  sems[p][0])
            dt = pltpu.async_copy(tail_hbm.at[tidx.at[c]], tbuf.at[p],
                                  sems[p][1])
            dr = pltpu.async_copy(rel_hbm.at[ridx.at[c]], rbuf.at[p],
                                  sems[p][2])
            return (dh, dt, dr)

        pend = fire(0, 0)
        for c in range(NCHUNK):
            p = c % 2
            for d in pend:
                d.wait()
            if c + 1 < NCHUNK:
                pend = fire(c + 1, 1 - p)
            hb, tb, rb = hbuf.at[p], tbuf.at[p], rbuf.at[p]

            @plsc.parallel_loop(0, CHUNK, unroll=1)
            def row_fn(i, hb=hb, tb=tb, rb=rb):
                hv = [hb[i, pl.ds(s * 16, 16)] for s in range(NSUB)]
                tv = [tb[i, pl.ds(s * 16, 16)] for s in range(NSUB)]
                rv = [rb[i, pl.ds(s * 16, 16)] for s in range(NSUB)]
                ih = _rsqrt(jnp.sum(_treesum([v * v for v in hv])))
                it = _rsqrt(jnp.sum(_treesum([v * v for v in tv])))
                ir = _rsqrt(jnp.sum(_treesum([v * v for v in rv])))
                ihv = jnp.full((16,), ih, jnp.float32)
                itv = jnp.full((16,), it, jnp.float32)
                irv = jnp.full((16,), ir, jnp.float32)
                acc = _treesum([
                    jnp.abs(hv[s] * ihv + rv[s] * irv - tv[s] * itv)
                    for s in range(NSUB)
                ])
                pb_s[i, pl.ds(0, 16)] = acc

            def grp(b, _, c=c):
                sc = zero
                for j in range(16):
                    jv = jnp.full((16,), j, jnp.int32)
                    sc = sc + plsc.load_gather(pb_s, [b * 16 + lane, jv])
                score[c, pl.ds(b * 16, 16)] = sc
                return 0

            lax.fori_loop(0, NBLK, grp, 0)

        pltpu.sync_copy(score, out_hbm.at[wid])

    return trans_score


_sc_score = _make_kernel()


def kernel(batch_h, batch_t, batch_r, tail_emb, rel_emb):
    h = batch_h.astype(jnp.int32).reshape(NW, NCHUNK, CHUNK)
    t = batch_t.astype(jnp.int32).reshape(NW, NCHUNK, CHUNK)
    r = batch_r.astype(jnp.int32).reshape(NW, NCHUNK, CHUNK)
    out = _sc_score(h, t, r, tail_emb, rel_emb)
    return out.reshape(-1)
